# baseline (speedup 1.0000x reference)
import jax
import jax.numpy as jnp
from jax import lax
from jax.experimental import pallas as pl
from jax.experimental.pallas import tpu as pltpu
from jax.experimental.pallas import tpu_sc as plsc

N_NODES = 10000
N_EDGES = 160000
D = 256
SLOPE = 0.2

NPAD = 10240
TILES = 32
RPT = NPAD // TILES
K = 128
CAP = 272
SROW = 16
SGE = SROW * 128
SG = 80
EPAD = SG * SGE


def _tc_body(x_ref, w_ref, b_ref, a1_ref, a2_ref, h_ref, at_ref, an_ref):
    dn = (((1,), (1,)), ((), ()))
    h = lax.dot_general(x_ref[...], w_ref[...], dn,
                        preferred_element_type=jnp.float32,
                        precision=lax.Precision.HIGHEST)
    h = h + b_ref[...]
    h_ref[...] = h
    at_ref[...] = lax.dot_general(h, a1_ref[...], dn,
                                  preferred_element_type=jnp.float32,
                                  precision=lax.Precision.HIGHEST)
    an_ref[...] = lax.dot_general(h, a2_ref[...], dn,
                                  preferred_element_type=jnp.float32,
                                  precision=lax.Precision.HIGHEST)


def _tc_stage(x, w, b, a1, a2):
    mb = 1024
    grid = (NPAD // mb,)
    return pl.pallas_call(
        _tc_body,
        grid=grid,
        in_specs=[
            pl.BlockSpec((mb, D), lambda i: (i, 0)),
            pl.BlockSpec((D, D), lambda i: (0, 0)),
            pl.BlockSpec((1, D), lambda i: (0, 0)),
            pl.BlockSpec((1, D), lambda i: (0, 0)),
            pl.BlockSpec((1, D), lambda i: (0, 0)),
        ],
        out_specs=[
            pl.BlockSpec((mb, D), lambda i: (i, 0)),
            pl.BlockSpec((mb, 1), lambda i: (i, 0)),
            pl.BlockSpec((mb, 1), lambda i: (i, 0)),
        ],
        out_shape=[
            jax.ShapeDtypeStruct((NPAD, D), jnp.float32),
            jax.ShapeDtypeStruct((NPAD, 1), jnp.float32),
            jax.ShapeDtypeStruct((NPAD, 1), jnp.float32),
        ],
    )(x, w, b, a1, a2)


def _sc_body(h_hbm, at_hbm, an_hbm, cvec_hbm, t_hbm, n_hbm, out_hbm,
             ti, ni, ct, cn, ct2, cn2, tlv, wbuf, atb, anb, rows, dn, cbuf,
             acc,
             sem_t, sem_n, sem_a1, sem_a2, sem_r):
    c = lax.axis_index("c")
    s = lax.axis_index("s")
    tid = s * 2 + c
    lo = tid * RPT

    pltpu.sync_copy(cvec_hbm, cbuf)

    zf = jnp.zeros((16,), jnp.float32)
    zi = jnp.zeros((16,), jnp.int32)
    iota16 = lax.iota(jnp.int32, 16)

    def _zacc(r, _):
        for cc in range(16):
            acc[r, pl.ds(cc * 16, 16)] = zf
        return 0

    lax.fori_loop(0, RPT, _zacc, 0)
    for g in range(RPT // 16):
        dn[pl.ds(g * 16, 16)] = zf
    for g in range(CAP // 16):
        ct[pl.ds(g * 16, 16)] = zi
        cn[pl.ds(g * 16, 16)] = zi

    cv = cbuf[...]

    def _issue(cntv, full):
        for g in range(K // 16):
            sl = pl.ds(g * 16, 16)
            ct2[sl] = ct[sl]
            cn2[sl] = cn[sl]
        pltpu.async_copy(at_hbm.at[ct2], atb, sem_a1)
        pltpu.async_copy(an_hbm.at[cn2], anb, sem_a2)
        pltpu.async_copy(h_hbm.at[cn2], rows, sem_r)
        if full:
            for g in range(K // 16):
                sl_src = pl.ds(K + g * 16, 16)
                sl_dst = pl.ds(g * 16, 16)
                ct[sl_dst] = ct[sl_src]
                cn[sl_dst] = cn[sl_src]
            return cntv - K
        return jnp.zeros((16,), jnp.int32)

    def _resolve(nvalid):
        pltpu.make_async_copy(at_hbm.at[ct2], atb, sem_a1).wait()
        pltpu.make_async_copy(an_hbm.at[cn2], anb, sem_a2).wait()
        for c8 in range(K // 16):
            sl = pl.ds(c8 * 16, 16)
            e = atb[sl] + anb[sl]
            e = jnp.maximum(e, SLOPE * e)
            w = jnp.exp(e - cv)
            w = jnp.where(c8 * 16 + iota16 < nvalid, w, 0.0)
            wbuf[sl] = w
            tv = ct2[sl] - lo
            tlv[sl] = jnp.minimum(jnp.maximum(tv, 0), RPT - 1)
        for c8 in range(K // 16):
            sl = pl.ds(c8 * 16, 16)
            plsc.addupdate_scatter(dn, [tlv[sl]], wbuf[sl])
        pltpu.make_async_copy(h_hbm.at[cn2], rows, sem_r).wait()

        def _scale(r, _):
            ridx = jnp.full((16,), r, jnp.int32)
            wspl = plsc.load_gather(wbuf, [ridx])
            tspl = plsc.load_gather(tlv, [ridx])
            for cc in range(16):
                sl2 = pl.ds(cc * 16, 16)
                ci = cc * 16 + iota16
                cur = plsc.load_gather(acc, [tspl, ci])
                plsc.store_scatter(acc, [tspl, ci],
                                   cur + rows[r, sl2] * wspl)
            return 0

        lax.fori_loop(0, K, _scale, 0)

    def _row_scan(jr, carry):
        p, cntv, pend = carry
        for c8 in range(8):
            sl = pl.ds(c8 * 16, 16)
            t16 = ti[p, jr, sl]
            n16 = ni[p, jr, sl]
            m = (t16 >= lo) & (t16 < lo + RPT)
            ts, ns, _ = plsc.sort_key_val(t16, n16, mask=m)
            idx = cntv + iota16
            plsc.store_scatter(ct, [idx], ts)
            plsc.store_scatter(cn, [idx], ns)
            pc = plsc.all_reduce_population_count(m)
            cntv = cntv + pc
        def _trigger(cv_, pend_):
            lax.cond(pend_ > 0, lambda: _resolve(jnp.full((16,), K, jnp.int32)),
                     lambda: None)
            return _issue(cv_, True), jnp.int32(1)

        cntv, pend = lax.cond(cntv[0] >= K, _trigger,
                              lambda cv_, pend_: (cv_, pend_), cntv, pend)
        return (p, cntv, pend)

    def _stage(sg, p):
        pltpu.async_copy(t_hbm.at[sg], ti.at[p], sem_t)
        pltpu.async_copy(n_hbm.at[sg], ni.at[p], sem_n)

    def _stage_wait(sg, p):
        pltpu.make_async_copy(t_hbm.at[sg], ti.at[p], sem_t).wait()
        pltpu.make_async_copy(n_hbm.at[sg], ni.at[p], sem_n).wait()

    def _sg(sg, carry):
        cntv, pend = carry
        p = jnp.bitwise_and(sg, 1)
        lax.cond(sg + 1 < SG,
                 lambda: _stage(sg + 1, 1 - p), lambda: None)
        _stage_wait(sg, p)
        _, cntv, pend = lax.fori_loop(0, SROW, _row_scan, (p, cntv, pend))
        return (cntv, pend)

    _stage(0, 0)
    cntv, pend = lax.fori_loop(0, SG, _sg,
                               (jnp.zeros((16,), jnp.int32), jnp.int32(0)))
    lax.cond(pend > 0, lambda: _resolve(jnp.full((16,), K, jnp.int32)),
             lambda: None)
    _issue(cntv, False)
    _resolve(cntv)

    def _hstage(j5, p):
        pltpu.async_copy(h_hbm.at[pl.ds(lo + j5 * 16, 16)],
                         rows.at[pl.ds(p * 16, 16)], sem_r)

    for g in range(RPT // 16):
        sl = pl.ds(g * 16, 16)
        dn[sl] = 1.0 / jnp.maximum(dn[sl], 1e-30)

    def _norm(j5, _):
        p = jnp.bitwise_and(j5, 1)
        lax.cond(j5 + 1 < RPT // 16,
                 lambda: _hstage(j5 + 1, 1 - p), lambda: None)
        pltpu.make_async_copy(h_hbm.at[pl.ds(lo, 16)],
                              rows.at[pl.ds(0, 16)], sem_r).wait()
        for r16 in range(16):
            r = j5 * 16 + r16
            rec = plsc.load_gather(dn, [jnp.full((16,), r, jnp.int32)])
            for cc in range(16):
                sl2 = pl.ds(cc * 16, 16)
                acc[r, sl2] = acc[r, sl2] * rec + rows[p * 16 + r16, sl2]
        return 0

    _hstage(0, 0)
    lax.fori_loop(0, RPT // 16, _norm, 0)
    pltpu.sync_copy(acc, out_hbm.at[pl.ds(lo, RPT)])


def _sc_stage(h, at_flat, an_flat, cvec, t3, n3):
    mesh = plsc.VectorSubcoreMesh(core_axis_name="c", subcore_axis_name="s")
    f = pl.kernel(
        _sc_body,
        out_type=jax.ShapeDtypeStruct((NPAD, D), jnp.float32),
        mesh=mesh,
        compiler_params=pltpu.CompilerParams(needs_layout_passes=False),
        scratch_types=[
            pltpu.VMEM((2, SROW, 128), jnp.int32),
            pltpu.VMEM((2, SROW, 128), jnp.int32),
            pltpu.VMEM((CAP,), jnp.int32),
            pltpu.VMEM((CAP,), jnp.int32),
            pltpu.VMEM((K,), jnp.int32),
            pltpu.VMEM((K,), jnp.int32),
            pltpu.VMEM((K,), jnp.int32),
            pltpu.VMEM((K,), jnp.float32),
            pltpu.VMEM((K,), jnp.float32),
            pltpu.VMEM((K,), jnp.float32),
            pltpu.VMEM((K, D), jnp.float32),
            pltpu.VMEM((RPT,), jnp.float32),
            pltpu.VMEM((16,), jnp.float32),
            pltpu.VMEM((RPT, D), jnp.float32),
            pltpu.SemaphoreType.DMA,
            pltpu.SemaphoreType.DMA,
            pltpu.SemaphoreType.DMA,
            pltpu.SemaphoreType.DMA,
            pltpu.SemaphoreType.DMA,
        ],
    )
    return f(h, at_flat, an_flat, cvec, t3, n3)


@jax.jit
def kernel(node_features, edge_index, w_weight, w_bias, attn_weight):
    x = jnp.pad(node_features, ((0, NPAD - N_NODES), (0, 0)))
    b = w_bias.reshape(1, D)
    a1 = attn_weight[:, :D]
    a2 = attn_weight[:, D:]
    h, at2, an2 = _tc_stage(x, w_weight, b, a1, a2)
    at_flat = at2.reshape(NPAD)
    an_flat = an2.reshape(NPAD)
    cval = jnp.max(at_flat) + jnp.max(an_flat)
    cvec = jnp.broadcast_to(cval, (16,)).astype(jnp.float32)

    ei = edge_index.astype(jnp.int32)
    t3 = jnp.pad(ei[0], (0, EPAD - N_EDGES),
                 constant_values=-1).reshape(SG, SROW, 128)
    n3 = jnp.pad(ei[1], (0, EPAD - N_EDGES)).reshape(SG, SROW, 128)

    out_full = _sc_stage(h, at_flat, an_flat, cvec, t3, n3)
    return out_full[:N_NODES]


# speedup vs baseline: 1.1805x; 1.1805x over previous
import jax
import jax.numpy as jnp
from jax import lax
from jax.experimental import pallas as pl
from jax.experimental.pallas import tpu as pltpu
from jax.experimental.pallas import tpu_sc as plsc

N_NODES = 10000
N_EDGES = 160000
D = 256
SLOPE = 0.2

NPAD = 10240
TILES = 32
RPT = NPAD // TILES
K = 128
CAP = 272
SROW = 16
SGE = SROW * 128
SG = 80
EPAD = SG * SGE


def _tc_body(x_ref, w_ref, b_ref, a1_ref, a2_ref, h_ref, at_ref, an_ref):
    dn = (((1,), (1,)), ((), ()))
    h = lax.dot_general(x_ref[...], w_ref[...], dn,
                        preferred_element_type=jnp.float32,
                        precision=lax.Precision.HIGHEST)
    h = h + b_ref[...]
    h_ref[...] = h
    at_ref[...] = lax.dot_general(h, a1_ref[...], dn,
                                  preferred_element_type=jnp.float32,
                                  precision=lax.Precision.HIGHEST)
    an_ref[...] = lax.dot_general(h, a2_ref[...], dn,
                                  preferred_element_type=jnp.float32,
                                  precision=lax.Precision.HIGHEST)


def _tc_stage(x, w, b, a1, a2):
    mb = 1024
    grid = (NPAD // mb,)
    return pl.pallas_call(
        _tc_body,
        grid=grid,
        in_specs=[
            pl.BlockSpec((mb, D), lambda i: (i, 0)),
            pl.BlockSpec((D, D), lambda i: (0, 0)),
            pl.BlockSpec((1, D), lambda i: (0, 0)),
            pl.BlockSpec((1, D), lambda i: (0, 0)),
            pl.BlockSpec((1, D), lambda i: (0, 0)),
        ],
        out_specs=[
            pl.BlockSpec((mb, D), lambda i: (i, 0)),
            pl.BlockSpec((mb, 1), lambda i: (i, 0)),
            pl.BlockSpec((mb, 1), lambda i: (i, 0)),
        ],
        out_shape=[
            jax.ShapeDtypeStruct((NPAD, D), jnp.float32),
            jax.ShapeDtypeStruct((NPAD, 1), jnp.float32),
            jax.ShapeDtypeStruct((NPAD, 1), jnp.float32),
        ],
    )(x, w, b, a1, a2)


def _sc_body(h_hbm, at_hbm, an_hbm, cvec_hbm, t_hbm, n_hbm, out_hbm,
             ti, ni, ct, cn, ct2, cn2, tlv, wbuf, atb, anb, rows, dn, cbuf,
             acc,
             sem_t, sem_n, sem_a1, sem_a2, sem_r):
    c = lax.axis_index("c")
    s = lax.axis_index("s")
    tid = s * 2 + c
    lo = tid * RPT

    pltpu.sync_copy(cvec_hbm, cbuf)

    zf = jnp.zeros((16,), jnp.float32)
    zi = jnp.zeros((16,), jnp.int32)
    iota16 = lax.iota(jnp.int32, 16)

    def _zacc(r, _):
        for cc in range(16):
            acc[r, pl.ds(cc * 16, 16)] = zf
        return 0

    lax.fori_loop(0, RPT, _zacc, 0)
    for g in range(RPT // 16):
        dn[pl.ds(g * 16, 16)] = zf
    for g in range(CAP // 16):
        ct[pl.ds(g * 16, 16)] = zi
        cn[pl.ds(g * 16, 16)] = zi

    cv = cbuf[...]

    def _issue(cntv, full):
        for g in range(K // 16):
            sl = pl.ds(g * 16, 16)
            pk = ct[sl]
            ct2[sl] = jnp.minimum(jnp.bitwise_and(pk, 16383), NPAD - 1)
            cn2[sl] = jnp.minimum(
                jnp.bitwise_and(jnp.right_shift(pk, 14), 16383), NPAD - 1)
        pltpu.async_copy(at_hbm.at[ct2], atb, sem_a1)
        pltpu.async_copy(an_hbm.at[cn2], anb, sem_a2)
        pltpu.async_copy(h_hbm.at[cn2], rows, sem_r)
        if full:
            for g in range(K // 16):
                ct[pl.ds(g * 16, 16)] = ct[pl.ds(K + g * 16, 16)]
            return cntv - K
        return jnp.zeros((16,), jnp.int32)

    def _resolve(nvalid):
        pltpu.make_async_copy(at_hbm.at[ct2], atb, sem_a1).wait()
        pltpu.make_async_copy(an_hbm.at[cn2], anb, sem_a2).wait()
        for c8 in range(K // 16):
            sl = pl.ds(c8 * 16, 16)
            e = atb[sl] + anb[sl]
            e = jnp.maximum(e, SLOPE * e)
            w = jnp.exp(e - cv)
            w = jnp.where(c8 * 16 + iota16 < nvalid, w, 0.0)
            wbuf[sl] = w
            tv = ct2[sl] - lo
            tlv[sl] = jnp.minimum(jnp.maximum(tv, 0), RPT - 1)
        for c8 in range(K // 16):
            sl = pl.ds(c8 * 16, 16)
            plsc.addupdate_scatter(dn, [tlv[sl]], wbuf[sl])
        pltpu.make_async_copy(h_hbm.at[cn2], rows, sem_r).wait()

        def _scale(r, _):
            ridx = jnp.full((16,), r, jnp.int32)
            wspl = plsc.load_gather(wbuf, [ridx])
            tspl = plsc.load_gather(tlv, [ridx])
            for cc in range(16):
                sl2 = pl.ds(cc * 16, 16)
                plsc.addupdate_scatter(
                    acc, [tspl, cc * 16 + iota16], rows[r, sl2] * wspl)
            return 0

        lax.fori_loop(0, K, _scale, 0)

    def _row_scan(jr, carry):
        p, cntv, pend = carry
        for c8 in range(8):
            sl = pl.ds(c8 * 16, 16)
            t16 = ti[p, jr, sl]
            n16 = ni[p, jr, sl]
            m = (t16 >= lo) & (t16 < lo + RPT)
            pk = t16 + jnp.left_shift(n16, 14)
            _, ps, _ = plsc.sort_key_val(pk, pk, mask=m)
            plsc.store_scatter(ct, [cntv + iota16], ps)
            pc = plsc.all_reduce_population_count(m)
            cntv = cntv + pc
        def _trigger(cv_, pend_):
            lax.cond(pend_ > 0, lambda: _resolve(jnp.full((16,), K, jnp.int32)),
                     lambda: None)
            return _issue(cv_, True), jnp.int32(1)

        cntv, pend = lax.cond(cntv[0] >= K, _trigger,
                              lambda cv_, pend_: (cv_, pend_), cntv, pend)
        return (p, cntv, pend)

    def _stage(sg, p):
        pltpu.async_copy(t_hbm.at[sg], ti.at[p], sem_t)
        pltpu.async_copy(n_hbm.at[sg], ni.at[p], sem_n)

    def _stage_wait(sg, p):
        pltpu.make_async_copy(t_hbm.at[sg], ti.at[p], sem_t).wait()
        pltpu.make_async_copy(n_hbm.at[sg], ni.at[p], sem_n).wait()

    def _sg(sg, carry):
        cntv, pend = carry
        p = jnp.bitwise_and(sg, 1)
        lax.cond(sg + 1 < SG,
                 lambda: _stage(sg + 1, 1 - p), lambda: None)
        _stage_wait(sg, p)
        _, cntv, pend = lax.fori_loop(0, SROW, _row_scan, (p, cntv, pend))
        return (cntv, pend)

    _stage(0, 0)
    cntv, pend = lax.fori_loop(0, SG, _sg,
                               (jnp.zeros((16,), jnp.int32), jnp.int32(0)))
    lax.cond(pend > 0, lambda: _resolve(jnp.full((16,), K, jnp.int32)),
             lambda: None)
    _issue(cntv, False)
    _resolve(cntv)

    def _hstage(j5, p):
        pltpu.async_copy(h_hbm.at[pl.ds(lo + j5 * 16, 16)],
                         rows.at[pl.ds(p * 16, 16)], sem_r)

    for g in range(RPT // 16):
        sl = pl.ds(g * 16, 16)
        dn[sl] = 1.0 / jnp.maximum(dn[sl], 1e-30)

    def _norm(j5, _):
        p = jnp.bitwise_and(j5, 1)
        lax.cond(j5 + 1 < RPT // 16,
                 lambda: _hstage(j5 + 1, 1 - p), lambda: None)
        pltpu.make_async_copy(h_hbm.at[pl.ds(lo, 16)],
                              rows.at[pl.ds(0, 16)], sem_r).wait()
        for r16 in range(16):
            r = j5 * 16 + r16
            rec = plsc.load_gather(dn, [jnp.full((16,), r, jnp.int32)])
            for cc in range(16):
                sl2 = pl.ds(cc * 16, 16)
                acc[r, sl2] = acc[r, sl2] * rec + rows[p * 16 + r16, sl2]
        return 0

    _hstage(0, 0)
    lax.fori_loop(0, RPT // 16, _norm, 0)
    pltpu.sync_copy(acc, out_hbm.at[pl.ds(lo, RPT)])


def _sc_stage(h, at_flat, an_flat, cvec, t3, n3):
    mesh = plsc.VectorSubcoreMesh(core_axis_name="c", subcore_axis_name="s")
    f = pl.kernel(
        _sc_body,
        out_type=jax.ShapeDtypeStruct((NPAD, D), jnp.float32),
        mesh=mesh,
        compiler_params=pltpu.CompilerParams(needs_layout_passes=False),
        scratch_types=[
            pltpu.VMEM((2, SROW, 128), jnp.int32),
            pltpu.VMEM((2, SROW, 128), jnp.int32),
            pltpu.VMEM((CAP,), jnp.int32),
            pltpu.VMEM((CAP,), jnp.int32),
            pltpu.VMEM((K,), jnp.int32),
            pltpu.VMEM((K,), jnp.int32),
            pltpu.VMEM((K,), jnp.int32),
            pltpu.VMEM((K,), jnp.float32),
            pltpu.VMEM((K,), jnp.float32),
            pltpu.VMEM((K,), jnp.float32),
            pltpu.VMEM((K, D), jnp.float32),
            pltpu.VMEM((RPT,), jnp.float32),
            pltpu.VMEM((16,), jnp.float32),
            pltpu.VMEM((RPT, D), jnp.float32),
            pltpu.SemaphoreType.DMA,
            pltpu.SemaphoreType.DMA,
            pltpu.SemaphoreType.DMA,
            pltpu.SemaphoreType.DMA,
            pltpu.SemaphoreType.DMA,
        ],
    )
    return f(h, at_flat, an_flat, cvec, t3, n3)


@jax.jit
def kernel(node_features, edge_index, w_weight, w_bias, attn_weight):
    x = jnp.pad(node_features, ((0, NPAD - N_NODES), (0, 0)))
    b = w_bias.reshape(1, D)
    a1 = attn_weight[:, :D]
    a2 = attn_weight[:, D:]
    h, at2, an2 = _tc_stage(x, w_weight, b, a1, a2)
    at_flat = at2.reshape(NPAD)
    an_flat = an2.reshape(NPAD)
    cval = jnp.max(at_flat) + jnp.max(an_flat)
    cvec = jnp.broadcast_to(cval, (16,)).astype(jnp.float32)

    ei = edge_index.astype(jnp.int32)
    t3 = jnp.pad(ei[0], (0, EPAD - N_EDGES),
                 constant_values=-1).reshape(SG, SROW, 128)
    n3 = jnp.pad(ei[1], (0, EPAD - N_EDGES)).reshape(SG, SROW, 128)

    out_full = _sc_stage(h, at_flat, an_flat, cvec, t3, n3)
    return out_full[:N_NODES]
